# baseline (device time: 86187 ns/iter reference)
import functools

import jax
import jax.numpy as jnp
from jax import lax
from jax.experimental import pallas as pl
from jax.experimental.pallas import tpu as pltpu

N_DEV = 4
N_TOK = 2048
D = 1024
H = 1024
E_LOC = 8
CHUNK = N_TOK // N_DEV


def kernel(x, router_W, route_idx, expert_W):
    x_bf = x.astype(jnp.bfloat16)
    ew_bf = expert_W.astype(jnp.bfloat16)

    def body(x_ref, xbf_ref, rw_ref, idx_ref, ew_ref, out_ref,
             acc_ref, gn_ref, sendb_ref, recvb_ref, send_sems, recv_sems):
        my = lax.axis_index("i")
        right = lax.rem(my + 1, N_DEV)
        left = lax.rem(my + 3, N_DEV)

        bar = pltpu.get_barrier_semaphore()
        for nbr in (left, right):
            pl.semaphore_signal(bar, inc=1, device_id=(nbr,),
                                device_id_type=pl.DeviceIdType.MESH)
        pl.semaphore_wait(bar, 2)

        scores = jnp.dot(x_ref[:, :], rw_ref[:, :],
                         preferred_element_type=jnp.float32)
        smax = jnp.max(scores, axis=1, keepdims=True)
        p = jnp.exp(scores - smax)
        e0 = idx_ref[:, 0:1]
        e1 = idx_ref[:, 1:2]
        iota = lax.broadcasted_iota(jnp.int32, (N_TOK, 32), 1)
        g0 = jnp.sum(jnp.where(iota == e0, p, 0.0), axis=1, keepdims=True)
        g1 = jnp.sum(jnp.where(iota == e1, p, 0.0), axis=1, keepdims=True)
        gs = g0 + g1
        gn_ref[:, 0:1] = g0 / gs
        gn_ref[:, 1:2] = g1 / gs

        def partial_into_acc(c):
            r0 = c * CHUNK
            rows = pl.ds(r0, CHUNK)
            xb = xbf_ref[rows, :]
            e0c = idx_ref[rows, 0:1]
            e1c = idx_ref[rows, 1:2]
            g0c = gn_ref[rows, 0:1]
            g1c = gn_ref[rows, 1:2]
            for j in range(E_LOC):
                ej = my * E_LOC + j
                w = (jnp.where(e0c == ej, g0c, 0.0)
                     + jnp.where(e1c == ej, g1c, 0.0))
                y = jnp.dot(xb, ew_ref[j], preferred_element_type=jnp.float32)
                if j == 0:
                    acc_ref[:, :] = y * w
                else:
                    acc_ref[:, :] = acc_ref[:, :] + y * w

        partial_into_acc(lax.rem(my + 3, N_DEV))
        for s in range(N_DEV - 1):
            sendb_ref[s] = acc_ref[:, :].astype(jnp.bfloat16)
            rdma = pltpu.make_async_remote_copy(
                src_ref=sendb_ref.at[s],
                dst_ref=recvb_ref.at[s],
                send_sem=send_sems.at[s],
                recv_sem=recv_sems.at[s],
                device_id=(right,),
                device_id_type=pl.DeviceIdType.MESH,
            )
            rdma.start()
            partial_into_acc(lax.rem(my + 2 - s + N_DEV, N_DEV))
            rdma.wait()
            acc_ref[:, :] = acc_ref[:, :] + recvb_ref[s].astype(jnp.float32)

        out_ref[:, :] = acc_ref[:, :]

        @functools.partial(pl.run_scoped, sem2=pltpu.SemaphoreType.REGULAR)
        def _(sem2):
            for nbr in (left, right):
                pl.semaphore_signal(sem2, inc=1, device_id=(nbr,),
                                    device_id_type=pl.DeviceIdType.MESH)
            pl.semaphore_wait(sem2, 2)

    return pl.pallas_call(
        body,
        out_shape=jax.ShapeDtypeStruct((CHUNK, H), jnp.float32),
        in_specs=[pl.BlockSpec(memory_space=pltpu.VMEM)] * 5,
        out_specs=pl.BlockSpec(memory_space=pltpu.VMEM),
        scratch_shapes=[
            pltpu.VMEM((CHUNK, H), jnp.float32),
            pltpu.VMEM((N_TOK, 2), jnp.float32),
            pltpu.VMEM((N_DEV - 1, CHUNK, H), jnp.bfloat16),
            pltpu.VMEM((N_DEV - 1, CHUNK, H), jnp.bfloat16),
            pltpu.SemaphoreType.DMA((N_DEV - 1,)),
            pltpu.SemaphoreType.DMA((N_DEV - 1,)),
        ],
        compiler_params=pltpu.CompilerParams(collective_id=0),
    )(x, x_bf, router_W, route_idx, ew_bf)
